# baseline (speedup 1.0000x reference)
import jax
import jax.numpy as jnp
from jax.experimental import pallas as pl
from jax.experimental.pallas import tpu as pltpu

_NC = 1000
_SMOOTHING = 0.1
_CONF = 1.0 - _SMOOTHING
_EPS = _SMOOTHING / (_NC - 1)
_BLK = 2048


def _loss_block(pred_ref, tgt_ref, out_ref):
    i = pl.program_id(0)
    ng = pl.num_programs(0)
    x = pred_ref[...]
    t = tgt_ref[...]
    m = jnp.max(x, axis=1, keepdims=True)
    e = jnp.exp(x - m)
    col = jax.lax.broadcasted_iota(jnp.int32, (1, _NC), 1)
    w = jnp.where(col == t, _CONF, _EPS)
    ones_col = jnp.ones((_NC, 128), jnp.float32)
    s = jax.lax.dot(e, ones_col)[:, :1]
    wsum = jax.lax.dot(x * w, ones_col)[:, :1]
    lse = m + jnp.log(s)
    blk = jnp.sum(lse - wsum).reshape(1, 1)

    @pl.when(i == 0)
    def _init():
        out_ref[...] = jnp.zeros((1, 1), jnp.float32)

    out_ref[...] += blk

    @pl.when(i == ng - 1)
    def _final():
        out_ref[...] = out_ref[...] * (1.0 / (_BLK * ng))


def kernel(pred, target):
    n = target.shape[0]
    tgt2d = target.astype(jnp.int32).reshape(n, 1)
    grid = n // _BLK
    total = pl.pallas_call(
        _loss_block,
        grid=(grid,),
        in_specs=[
            pl.BlockSpec((_BLK, _NC), lambda i: (i, 0)),
            pl.BlockSpec((_BLK, 1), lambda i: (i, 0)),
        ],
        out_specs=pl.BlockSpec((1, 1), lambda i: (0, 0)),
        out_shape=jax.ShapeDtypeStruct((1, 1), jnp.float32),
    )(pred, tgt2d)
    return total[0, 0]


# speedup vs baseline: 1.0041x; 1.0041x over previous
import jax
import jax.numpy as jnp
from jax.experimental import pallas as pl
from jax.experimental.pallas import tpu as pltpu

_NC = 1000
_SMOOTHING = 0.1
_CONF = 1.0 - _SMOOTHING
_EPS = _SMOOTHING / (_NC - 1)
_BLK = 2048


def _loss_block(pred_ref, tgt_ref, out_ref):
    i = pl.program_id(0)
    ng = pl.num_programs(0)
    x = pred_ref[...]
    t = tgt_ref[...]
    m = jnp.max(x, axis=1, keepdims=True)
    s = jnp.sum(jnp.exp(x - m), axis=1, keepdims=True)
    lse = m + jnp.log(s)
    sum_pred = jnp.sum(x, axis=1, keepdims=True)
    col = jax.lax.broadcasted_iota(jnp.int32, (1, _NC), 1)
    p_t = jnp.sum(jnp.where(col == t, x, 0.0), axis=1, keepdims=True)
    blk = jnp.sum(lse - _EPS * sum_pred - (_CONF - _EPS) * p_t).reshape(1, 1)

    @pl.when(i == 0)
    def _init():
        out_ref[...] = jnp.zeros((1, 1), jnp.float32)

    out_ref[...] += blk

    @pl.when(i == ng - 1)
    def _final():
        out_ref[...] = out_ref[...] * (1.0 / (_BLK * ng))


def kernel(pred, target):
    n = target.shape[0]
    tgt2d = target.astype(jnp.int32).reshape(n, 1)
    grid = n // _BLK
    total = pl.pallas_call(
        _loss_block,
        grid=(grid,),
        in_specs=[
            pl.BlockSpec((_BLK, _NC), lambda i: (i, 0)),
            pl.BlockSpec((_BLK, 1), lambda i: (i, 0)),
        ],
        out_specs=pl.BlockSpec((1, 1), lambda i: (0, 0)),
        out_shape=jax.ShapeDtypeStruct((1, 1), jnp.float32),
    )(pred, tgt2d)
    return total[0, 0]
